# baseline (speedup 1.0000x reference)
import functools

import jax
import jax.numpy as jnp
from jax import lax
from jax.experimental import pallas as pl
from jax.experimental.pallas import tpu as pltpu
from jax.experimental.pallas import tpu_sc as plsc

B = 16384
D = 32
_NC, _NS = 2, 16
_NW = _NC * _NS
_BPW = B // _NW
_CH = 128
_NCH = _BPW // _CH
_IDX_ROWS = B // _CH


def _sc_body(aidx_h, tidx_h, iaidx_h, itidx_h, aeu_h, teu_h, aei_h, tei_h,
             wa_h, wb_h,
             au_out, tu_out, ai_out, ti_out,
             aidx, tidx, aiidx, tiidx, g0, g1, g2, g3, g4, g5, sem_a, sem_b):
    wid = lax.axis_index("s") * _NC + lax.axis_index("c")
    rbase = wid * _BPW
    ibase = wid * _NCH
    pltpu.sync_copy(aidx_h.at[pl.ds(ibase, _NCH)], aidx)
    pltpu.sync_copy(tidx_h.at[pl.ds(ibase, _NCH)], tidx)
    pltpu.sync_copy(iaidx_h.at[pl.ds(ibase, _NCH)], aiidx)
    pltpu.sync_copy(itidx_h.at[pl.ds(ibase, _NCH)], tiidx)

    copies_item = []
    copies_gate = []
    for c in range(_NCH):
        s = pl.ds(c * _CH, _CH)
        copies_item.append(pltpu.async_copy(aei_h.at[aiidx.at[c]], g0.at[s], sem_a))
        copies_item.append(pltpu.async_copy(tei_h.at[tiidx.at[c]], g1.at[s], sem_a))
        copies_gate.append(pltpu.async_copy(wa_h.at[aidx.at[c]], g2.at[s], sem_b))
        copies_gate.append(pltpu.async_copy(wa_h.at[tidx.at[c]], g3.at[s], sem_b))
        copies_gate.append(pltpu.async_copy(wb_h.at[aidx.at[c]], g4.at[s], sem_b))
        copies_gate.append(pltpu.async_copy(wb_h.at[tidx.at[c]], g5.at[s], sem_b))
    for d in copies_item:
        d.wait()
    pltpu.sync_copy(g0, ai_out.at[pl.ds(rbase, _BPW)])
    pltpu.sync_copy(g1, ti_out.at[pl.ds(rbase, _BPW)])

    copies_user = []
    for c in range(_NCH):
        s = pl.ds(c * _CH, _CH)
        copies_user.append(pltpu.async_copy(aeu_h.at[aidx.at[c]], g0.at[s], sem_a))
        copies_user.append(pltpu.async_copy(teu_h.at[tidx.at[c]], g1.at[s], sem_a))
    for d in copies_user:
        d.wait()
    for d in copies_gate:
        d.wait()

    def body(i, carry):
        for h in (0, 16):
            s = pl.ds(h, 16)
            a = g0[i, s]
            t = g1[i, s]
            au = g2[i, s] * a + (1.0 - g3[i, s]) * t
            tu = g4[i, s] * a + (1.0 - g5[i, s]) * t
            g2[i, s] = au
            g4[i, s] = tu
        return carry

    lax.fori_loop(0, _BPW, body, 0)
    pltpu.sync_copy(g2, au_out.at[pl.ds(rbase, _BPW)])
    pltpu.sync_copy(g4, tu_out.at[pl.ds(rbase, _BPW)])


_sc_gather = pl.kernel(
    _sc_body,
    out_type=[jax.ShapeDtypeStruct((B, D), jnp.float32)] * 4,
    mesh=plsc.VectorSubcoreMesh(core_axis_name="c", subcore_axis_name="s"),
    compiler_params=pltpu.CompilerParams(use_tc_tiling_on_sc=False),
    scratch_types=(
        [pltpu.VMEM((_NCH, _CH), jnp.int32)] * 4
        + [pltpu.VMEM((_BPW, D), jnp.float32)] * 6
        + [pltpu.SemaphoreType.DMA] * 2
    ),
)

_BT = 2048
_GRID = B // _BT


def _tc_body(ar_ref, tr_ref, au_ref, tu_ref, ai_ref, ti_ref,
             w1_ref, b1_ref, w2_ref, b2_ref, la_ref, lt_ref):
    i = pl.program_id(0)
    x = jnp.concatenate(
        [au_ref[...], tu_ref[...], ai_ref[...], ti_ref[...]], axis=1)
    h = jnp.maximum(
        jnp.dot(x, w1_ref[...], preferred_element_type=jnp.float32)
        + b1_ref[...], 0.0)
    y = jnp.maximum(
        jnp.dot(h, w2_ref[...], preferred_element_type=jnp.float32)
        + b2_ref[...], 0.0)
    a_s = jnp.maximum(jnp.sum(y[:, 0:32] * y[:, 64:96], axis=1), 1e-6)
    t_s = jnp.maximum(jnp.sum(y[:, 32:64] * y[:, 96:128], axis=1), 1e-6)
    da = a_s - ar_ref[0, 0, :]
    dt = t_s - tr_ref[0, 0, :]
    pa = jnp.sum(da * da) * (1.0 / B)
    pt = jnp.sum(dt * dt) * (1.0 / B)

    @pl.when(i == 0)
    def _():
        la_ref[0, 0] = 0.0
        lt_ref[0, 0] = 0.0

    la_ref[0, 0] += pa
    lt_ref[0, 0] += pt


_tc_dense = pl.pallas_call(
    _tc_body,
    grid=(_GRID,),
    in_specs=[
        pl.BlockSpec((1, 1, _BT), lambda i: (i, 0, 0)),
        pl.BlockSpec((1, 1, _BT), lambda i: (i, 0, 0)),
        pl.BlockSpec((_BT, D), lambda i: (i, 0)),
        pl.BlockSpec((_BT, D), lambda i: (i, 0)),
        pl.BlockSpec((_BT, D), lambda i: (i, 0)),
        pl.BlockSpec((_BT, D), lambda i: (i, 0)),
        pl.BlockSpec((4 * D, 8 * D), lambda i: (0, 0)),
        pl.BlockSpec((1, 8 * D), lambda i: (0, 0)),
        pl.BlockSpec((8 * D, 4 * D), lambda i: (0, 0)),
        pl.BlockSpec((1, 4 * D), lambda i: (0, 0)),
    ],
    out_specs=[
        pl.BlockSpec(memory_space=pltpu.SMEM),
        pl.BlockSpec(memory_space=pltpu.SMEM),
    ],
    out_shape=[jax.ShapeDtypeStruct((1, 1), jnp.float32)] * 2,
)


def _block_diag(ws):
    d_in, d_out = ws[0].shape
    full = jnp.zeros((4 * d_in, 4 * d_out), dtype=jnp.float32)
    for i, w in enumerate(ws):
        full = full.at[i * d_in:(i + 1) * d_in, i * d_out:(i + 1) * d_out].set(w)
    return full


def kernel(ausers, aitems, aratings, tusers, titems, tratings, params):
    p = params
    au2 = ausers.reshape(_IDX_ROWS, _CH)
    tu2 = tusers.reshape(_IDX_ROWS, _CH)
    ai2 = aitems.reshape(_IDX_ROWS, _CH)
    ti2 = titems.reshape(_IDX_ROWS, _CH)
    au_pre, tu_pre, ai_e, ti_e = _sc_gather(
        au2, tu2, ai2, ti2,
        p["a_emb_user"], p["t_emb_user"], p["a_emb_item"], p["t_emb_item"],
        p["W_a"], p["W_b"])

    mlps = [p["mlp_a_users"], p["mlp_t_users"], p["mlp_a_items"], p["mlp_t_items"]]
    w1 = _block_diag([m["W1"] for m in mlps])
    b1 = jnp.concatenate([m["b1"] for m in mlps]).reshape(1, 8 * D)
    w2 = _block_diag([m["W2"] for m in mlps])
    b2 = jnp.concatenate([m["b2"] for m in mlps]).reshape(1, 4 * D)

    ar3 = aratings.astype(jnp.float32).reshape(_GRID, 1, _BT)
    tr3 = tratings.astype(jnp.float32).reshape(_GRID, 1, _BT)
    la, lt = _tc_dense(ar3, tr3, au_pre, tu_pre, ai_e, ti_e, w1, b1, w2, b2)
    return (la[0, 0], lt[0, 0])
